# baseline (speedup 1.0000x reference)
import jax
import jax.numpy as jnp
from jax import lax
from jax.experimental import pallas as pl
from jax.experimental.pallas import tpu as pltpu
from jax.experimental.pallas import tpu_sc as plsc

VOCAB = 1000000
DIM = 16
BATCH = 16384
CTX = 20

NC = 2
NS = 16
NW = NC * NS
B_PER_W = BATCH // NW
CB = 32
NCHUNK = B_PER_W // CB
PAIRS = CB * CTX
GSLICE = 128


def _body(frow_hbm, fsub_hbm, crow_hbm, csub_hbm, win_hbm, wout_hbm, out_hbm,
          idx_f, sub_f, idx_c, sub_c, frows, crows, out_v, sem):
    wid = lax.axis_index("s") * NC + lax.axis_index("c")
    iota16 = lax.iota(jnp.int32, 16)

    def chunk_body(c, carry):
        chunk = wid * NCHUNK + c
        pltpu.sync_copy(frow_hbm.at[pl.ds(chunk * CB, CB)], idx_f)
        pltpu.sync_copy(fsub_hbm.at[pl.ds(chunk * CB, CB)], sub_f)
        pltpu.sync_copy(crow_hbm.at[pl.ds(chunk * PAIRS, PAIRS)], idx_c)
        pltpu.sync_copy(csub_hbm.at[pl.ds(chunk * PAIRS, PAIRS)], sub_c)
        copies = [pltpu.async_copy(win_hbm.at[idx_f], frows, sem)]
        for j in range(PAIRS // GSLICE):
            copies.append(pltpu.async_copy(
                wout_hbm.at[idx_c.at[pl.ds(j * GSLICE, GSLICE)]],
                crows.at[pl.ds(j * GSLICE, GSLICE)], sem))
        for cp in copies:
            cp.wait()

        def g_body(g, carry2):
            bvec = g * 16 + iota16
            fcol = plsc.load_gather(sub_f, [bvec])
            fcols = [plsc.load_gather(frows, [bvec, fcol + d])
                     for d in range(DIM)]
            base = bvec * CTX

            def l_body(l, carry3):
                pvec = base + l
                ccol = plsc.load_gather(sub_c, [pvec])
                acc = jnp.zeros((16,), jnp.float32)
                for d in range(DIM):
                    cv = plsc.load_gather(crows, [pvec, ccol + d])
                    acc = acc + cv * fcols[d]
                plsc.store_scatter(out_v, [pvec], acc)
                return carry3

            lax.fori_loop(0, CTX, l_body, 0)
            return carry2

        lax.fori_loop(0, CB // 16, g_body, 0)
        pltpu.sync_copy(out_v, out_hbm.at[pl.ds(chunk * PAIRS, PAIRS)])
        return carry

    lax.fori_loop(0, NCHUNK, chunk_body, 0)


def kernel(focus_item_batch, context_items_batch, W_in, W_out):
    focus = focus_item_batch.reshape(BATCH).astype(jnp.int32)
    ctx = context_items_batch.reshape(BATCH * CTX).astype(jnp.int32)
    frow = focus >> 3
    fsub = (focus & 7) * DIM
    crow = ctx >> 3
    csub = (ctx & 7) * DIM
    win8 = W_in.reshape(VOCAB // 8, 128)
    wout8 = W_out.reshape(VOCAB // 8, 128)
    run = pl.kernel(
        _body,
        out_type=jax.ShapeDtypeStruct((BATCH * CTX,), jnp.float32),
        mesh=plsc.VectorSubcoreMesh(core_axis_name="c", subcore_axis_name="s"),
        compiler_params=pltpu.CompilerParams(
            needs_layout_passes=False, use_tc_tiling_on_sc=True),
        scratch_types=[
            pltpu.VMEM((CB,), jnp.int32),
            pltpu.VMEM((CB,), jnp.int32),
            pltpu.VMEM((PAIRS,), jnp.int32),
            pltpu.VMEM((PAIRS,), jnp.int32),
            pltpu.VMEM((CB, 128), jnp.float32),
            pltpu.VMEM((PAIRS, 128), jnp.float32),
            pltpu.VMEM((PAIRS,), jnp.float32),
            pltpu.SemaphoreType.DMA,
        ],
    )
    out = run(frow, fsub, crow, csub, win8, wout8)
    return out.reshape(BATCH, CTX)
